# baseline (device time: 844292 ns/iter reference)
import jax
import jax.numpy as jnp
from jax import lax
from jax.experimental import pallas as pl
from jax.experimental.pallas import tpu as pltpu

N_DEV = 32
M_PER = 128
SLOTS = 2
H_R = N_DEV // 2
H_L = N_DEV - 1 - H_R


def kernel(x, w_mat):
    m_per, k = x.shape
    _, n_per = w_mat.shape
    assert m_per == M_PER

    def body(x_ref, w_ref, out_ref,
             rbuf, lbuf,
             r_send, r_recv, l_send, l_recv,
             r_credit, l_credit,
             amax_buf, amax_all, a_send, a_recv):
        me = lax.axis_index("i")
        left = jnp.mod(me - 1, N_DEV)
        right = jnp.mod(me + 1, N_DEV)

        barrier = pltpu.get_barrier_semaphore()
        for nbr in (left, right):
            pl.semaphore_signal(barrier, inc=1, device_id=(nbr,),
                                device_id_type=pl.DeviceIdType.MESH)
        pl.semaphore_wait(barrier, 2)

        def gemm(chunk):
            return lax.dot_general(chunk, w_ref[...],
                                   (((1,), (0,)), ((), ())),
                                   precision=lax.Precision.HIGHEST,
                                   preferred_element_type=jnp.float32)

        own = gemm(x_ref[...])
        out_ref[pl.ds(me * M_PER, M_PER), :] = own
        running = jnp.max(jnp.abs(own))

        for h in range(max(H_R, H_L)):
            rdma_r = rdma_l = None
            if h < H_R:
                if h >= SLOTS:
                    pl.semaphore_wait(r_credit, 1)
                src = x_ref if h == 0 else rbuf.at[(h - 1) % SLOTS]
                rdma_r = pltpu.make_async_remote_copy(
                    src_ref=src, dst_ref=rbuf.at[h % SLOTS],
                    send_sem=r_send.at[h], recv_sem=r_recv.at[h],
                    device_id=(right,), device_id_type=pl.DeviceIdType.MESH)
                rdma_r.start()
            if h < H_L:
                if h >= SLOTS:
                    pl.semaphore_wait(l_credit, 1)
                src = x_ref if h == 0 else lbuf.at[(h - 1) % SLOTS]
                rdma_l = pltpu.make_async_remote_copy(
                    src_ref=src, dst_ref=lbuf.at[h % SLOTS],
                    send_sem=l_send.at[h], recv_sem=l_recv.at[h],
                    device_id=(left,), device_id_type=pl.DeviceIdType.MESH)
                rdma_l.start()
            if rdma_r is not None:
                rdma_r.wait_recv()
                rdma_r.wait_send()
                if 1 <= h <= H_R - SLOTS:
                    pl.semaphore_signal(r_credit, inc=1, device_id=(left,),
                                        device_id_type=pl.DeviceIdType.MESH)
                blk = gemm(rbuf[h % SLOTS])
                origin = jnp.mod(me - h - 1, N_DEV)
                out_ref[pl.ds(origin * M_PER, M_PER), :] = blk
                running = jnp.maximum(running, jnp.max(jnp.abs(blk)))
            if rdma_l is not None:
                rdma_l.wait_recv()
                rdma_l.wait_send()
                if 1 <= h <= H_L - SLOTS:
                    pl.semaphore_signal(l_credit, inc=1, device_id=(right,),
                                        device_id_type=pl.DeviceIdType.MESH)
                blk = gemm(lbuf[h % SLOTS])
                origin = jnp.mod(me + h + 1, N_DEV)
                out_ref[pl.ds(origin * M_PER, M_PER), :] = blk
                running = jnp.maximum(running, jnp.max(jnp.abs(blk)))

        amax_buf[...] = jnp.full((8, 128), running, jnp.float32)
        rdmas = []
        for p in range(1, N_DEV):
            peer = jnp.mod(me + p, N_DEV)
            rd = pltpu.make_async_remote_copy(
                src_ref=amax_buf, dst_ref=amax_all.at[p - 1],
                send_sem=a_send.at[p - 1], recv_sem=a_recv.at[p - 1],
                device_id=(peer,), device_id_type=pl.DeviceIdType.MESH)
            rd.start()
            rdmas.append(rd)
        for rd in rdmas:
            rd.wait_send()
        for rd in rdmas:
            rd.wait_recv()
        total = jnp.maximum(running, jnp.max(amax_all[...]))

        scale = total / 448.0
        q = (out_ref[...] / scale).astype(jnp.float8_e4m3fn)
        out_ref[...] = q.astype(jnp.float32) * scale

    return pl.pallas_call(
        body,
        out_shape=jax.ShapeDtypeStruct((N_DEV * m_per, n_per), jnp.float32),
        in_specs=[pl.BlockSpec(memory_space=pltpu.VMEM),
                  pl.BlockSpec(memory_space=pltpu.VMEM)],
        out_specs=pl.BlockSpec(memory_space=pltpu.VMEM),
        scratch_shapes=[
            pltpu.VMEM((SLOTS, M_PER, k), jnp.float32),
            pltpu.VMEM((SLOTS, M_PER, k), jnp.float32),
            pltpu.SemaphoreType.DMA((H_R,)),
            pltpu.SemaphoreType.DMA((H_R,)),
            pltpu.SemaphoreType.DMA((H_L,)),
            pltpu.SemaphoreType.DMA((H_L,)),
            pltpu.SemaphoreType.REGULAR,
            pltpu.SemaphoreType.REGULAR,
            pltpu.VMEM((8, 128), jnp.float32),
            pltpu.VMEM((N_DEV - 1, 8, 128), jnp.float32),
            pltpu.SemaphoreType.DMA((N_DEV - 1,)),
            pltpu.SemaphoreType.DMA((N_DEV - 1,)),
        ],
        compiler_params=pltpu.CompilerParams(collective_id=0),
    )(x, w_mat)


# device time: 532168 ns/iter; 1.5865x vs baseline; 1.5865x over previous
import numpy as np

import jax
import jax.numpy as jnp
from jax import lax
from jax.experimental import pallas as pl
from jax.experimental.pallas import tpu as pltpu

N_DEV = 32
M_PER = 128
SLOTS = 4
H_R = N_DEV // 2
H_L = N_DEV - 1 - H_R


_COORD_TO_LOGICAL = {}
_l = 0
for _z in range(4):
    for _y in range(4):
        _xs = (1, 0) if _y % 2 else (0, 1)
        for _x in _xs:
            _COORD_TO_LOGICAL[(_x, _y, _z)] = _l
            _l += 1

_PATH_YZ = []
for _zi, _z in enumerate(range(4)):
    _ys = range(4) if _z % 2 == 0 else range(3, -1, -1)
    for _y in _ys:
        _PATH_YZ.append((_y, _z))
_RING_COORDS = [(0, y, z) for (y, z) in _PATH_YZ] + \
               [(1, y, z) for (y, z) in reversed(_PATH_YZ)]

RING_ORDER = np.array([_COORD_TO_LOGICAL[c] for c in _RING_COORDS],
                      dtype=np.int32)
RING_POS = np.argsort(RING_ORDER).astype(np.int32)


def kernel(x, w_mat):
    m_per, k = x.shape
    _, n_per = w_mat.shape
    assert m_per == M_PER

    me = lax.axis_index("i")
    ring_order = jnp.asarray(RING_ORDER)
    r_idx = jnp.asarray(RING_POS)[me]
    right_id = ring_order[jnp.mod(r_idx + 1, N_DEV)].reshape(1)
    left_id = ring_order[jnp.mod(r_idx - 1, N_DEV)].reshape(1)
    origins_r = ring_order[jnp.mod(r_idx - 1 - jnp.arange(H_R), N_DEV)]
    origins_l = ring_order[jnp.mod(r_idx + 1 + jnp.arange(H_L), N_DEV)]

    def body(right_ref, left_ref, org_r_ref, org_l_ref,
             x_ref, w_ref, out_ref,
             rbuf, lbuf,
             r_send, r_recv, l_send, l_recv,
             r_credit, l_credit,
             amax_buf, amax_all, a_send, a_recv):
        my = lax.axis_index("i")
        right = right_ref[0]
        left = left_ref[0]

        barrier = pltpu.get_barrier_semaphore()
        for nbr in (left, right):
            pl.semaphore_signal(barrier, inc=1, device_id=(nbr,),
                                device_id_type=pl.DeviceIdType.MESH)
        pl.semaphore_wait(barrier, 2)

        def gemm(chunk):
            return lax.dot_general(chunk, w_ref[...],
                                   (((1,), (0,)), ((), ())),
                                   precision=lax.Precision.HIGHEST,
                                   preferred_element_type=jnp.float32)

        own = gemm(x_ref[...])
        out_ref[pl.ds(my * M_PER, M_PER), :] = own
        running = jnp.max(jnp.abs(own))

        for h in range(max(H_R, H_L)):
            rdma_r = rdma_l = None
            if h < H_R:
                if h >= SLOTS:
                    pl.semaphore_wait(r_credit, 1)
                src = x_ref if h == 0 else rbuf.at[(h - 1) % SLOTS]
                rdma_r = pltpu.make_async_remote_copy(
                    src_ref=src, dst_ref=rbuf.at[h % SLOTS],
                    send_sem=r_send.at[h], recv_sem=r_recv.at[h],
                    device_id=(right,), device_id_type=pl.DeviceIdType.MESH)
                rdma_r.start()
            if h < H_L:
                if h >= SLOTS:
                    pl.semaphore_wait(l_credit, 1)
                src = x_ref if h == 0 else lbuf.at[(h - 1) % SLOTS]
                rdma_l = pltpu.make_async_remote_copy(
                    src_ref=src, dst_ref=lbuf.at[h % SLOTS],
                    send_sem=l_send.at[h], recv_sem=l_recv.at[h],
                    device_id=(left,), device_id_type=pl.DeviceIdType.MESH)
                rdma_l.start()
            if rdma_r is not None:
                rdma_r.wait_recv()
                rdma_r.wait_send()
                if 1 <= h <= H_R - SLOTS:
                    pl.semaphore_signal(r_credit, inc=1, device_id=(left,),
                                        device_id_type=pl.DeviceIdType.MESH)
                blk = gemm(rbuf[h % SLOTS])
                out_ref[pl.ds(org_r_ref[h] * M_PER, M_PER), :] = blk
                running = jnp.maximum(running, jnp.max(jnp.abs(blk)))
            if rdma_l is not None:
                rdma_l.wait_recv()
                rdma_l.wait_send()
                if 1 <= h <= H_L - SLOTS:
                    pl.semaphore_signal(l_credit, inc=1, device_id=(right,),
                                        device_id_type=pl.DeviceIdType.MESH)
                blk = gemm(lbuf[h % SLOTS])
                out_ref[pl.ds(org_l_ref[h] * M_PER, M_PER), :] = blk
                running = jnp.maximum(running, jnp.max(jnp.abs(blk)))

        amax_buf[...] = jnp.full((8, 128), running, jnp.float32)
        rdmas = []
        for p in range(1, N_DEV):
            peer = jnp.mod(my + p, N_DEV)
            rd = pltpu.make_async_remote_copy(
                src_ref=amax_buf, dst_ref=amax_all.at[p - 1],
                send_sem=a_send.at[p - 1], recv_sem=a_recv.at[p - 1],
                device_id=(peer,), device_id_type=pl.DeviceIdType.MESH)
            rd.start()
            rdmas.append(rd)
        for rd in rdmas:
            rd.wait_send()
        for rd in rdmas:
            rd.wait_recv()
        total = jnp.maximum(running, jnp.max(amax_all[...]))

        scale = total / 448.0
        q = (out_ref[...] / scale).astype(jnp.float8_e4m3fn)
        out_ref[...] = q.astype(jnp.float32) * scale

    smem = pl.BlockSpec(memory_space=pltpu.SMEM)
    return pl.pallas_call(
        body,
        out_shape=jax.ShapeDtypeStruct((N_DEV * m_per, n_per), jnp.float32),
        in_specs=[smem, smem, smem, smem,
                  pl.BlockSpec(memory_space=pltpu.VMEM),
                  pl.BlockSpec(memory_space=pltpu.VMEM)],
        out_specs=pl.BlockSpec(memory_space=pltpu.VMEM),
        scratch_shapes=[
            pltpu.VMEM((SLOTS, M_PER, k), jnp.float32),
            pltpu.VMEM((SLOTS, M_PER, k), jnp.float32),
            pltpu.SemaphoreType.DMA((H_R,)),
            pltpu.SemaphoreType.DMA((H_R,)),
            pltpu.SemaphoreType.DMA((H_L,)),
            pltpu.SemaphoreType.DMA((H_L,)),
            pltpu.SemaphoreType.REGULAR,
            pltpu.SemaphoreType.REGULAR,
            pltpu.VMEM((8, 128), jnp.float32),
            pltpu.VMEM((N_DEV - 1, 8, 128), jnp.float32),
            pltpu.SemaphoreType.DMA((N_DEV - 1,)),
            pltpu.SemaphoreType.DMA((N_DEV - 1,)),
        ],
        compiler_params=pltpu.CompilerParams(collective_id=0),
    )(right_id, left_id, origins_r, origins_l, x, w_mat)


# device time: 404623 ns/iter; 2.0866x vs baseline; 1.3152x over previous
import numpy as np

import jax
import jax.numpy as jnp
from jax import lax
from jax.experimental import pallas as pl
from jax.experimental.pallas import tpu as pltpu

N_DEV = 32
M_PER = 128
M_HALF = M_PER // 2
SLOTS = 4
H = 16


_COORD_TO_LOGICAL = {}
_l = 0
for _z in range(4):
    for _y in range(4):
        _xs = (1, 0) if _y % 2 else (0, 1)
        for _x in _xs:
            _COORD_TO_LOGICAL[(_x, _y, _z)] = _l
            _l += 1

_PATH_YZ = []
for _z in range(4):
    _ys = range(4) if _z % 2 == 0 else range(3, -1, -1)
    for _y in _ys:
        _PATH_YZ.append((_y, _z))
_RING_COORDS = [(0, y, z) for (y, z) in _PATH_YZ] + \
               [(1, y, z) for (y, z) in reversed(_PATH_YZ)]

RING_ORDER = np.array([_COORD_TO_LOGICAL[c] for c in _RING_COORDS],
                      dtype=np.int32)
RING_POS = np.argsort(RING_ORDER).astype(np.int32)


def kernel(x, w_mat):
    m_per, k = x.shape
    _, n_per = w_mat.shape
    assert m_per == M_PER

    me = lax.axis_index("i")
    ring_order = jnp.asarray(RING_ORDER)
    r_idx = jnp.asarray(RING_POS)[me]
    right_id = ring_order[jnp.mod(r_idx + 1, N_DEV)].reshape(1)
    left_id = ring_order[jnp.mod(r_idx - 1, N_DEV)].reshape(1)
    origins_r = ring_order[jnp.mod(r_idx - 1 - jnp.arange(H), N_DEV)]
    origins_l = ring_order[jnp.mod(r_idx + 1 + jnp.arange(H), N_DEV)]

    def body(right_ref, left_ref, org_r_ref, org_l_ref,
             x_ref, w_ref, out_ref,
             rbuf, lbuf,
             r_send, r_recv, l_send, l_recv,
             r_credit, l_credit,
             amax_buf, amax_all, a_send, a_recv):
        my = lax.axis_index("i")
        right = right_ref[0]
        left = left_ref[0]

        barrier = pltpu.get_barrier_semaphore()
        for nbr in (left, right):
            pl.semaphore_signal(barrier, inc=1, device_id=(nbr,),
                                device_id_type=pl.DeviceIdType.MESH)
        pl.semaphore_wait(barrier, 2)

        def gemm(chunk):
            return lax.dot_general(chunk, w_ref[...],
                                   (((1,), (0,)), ((), ())),
                                   precision=lax.Precision.HIGHEST,
                                   preferred_element_type=jnp.float32)

        running = jnp.float32(0.0)

        def do_block(chunk, origin, row_off, rows):
            nonlocal running
            blk = gemm(chunk)
            out_ref[pl.ds(origin * M_PER + row_off, rows), :] = blk
            running = jnp.maximum(running, jnp.max(jnp.abs(blk)))

        for h in range(H):
            if h >= SLOTS:
                pl.semaphore_wait(r_credit, 1)
                pl.semaphore_wait(l_credit, 1)
            if h == 0:
                src_r = src_l = x_ref
                dst_r = rbuf.at[0]
                dst_l = lbuf.at[0]
            elif h == H - 1:
                src_r = rbuf.at[(h - 1) % SLOTS, pl.ds(0, M_HALF)]
                src_l = lbuf.at[(h - 1) % SLOTS, pl.ds(M_HALF, M_HALF)]
                dst_r = rbuf.at[h % SLOTS, pl.ds(0, M_HALF)]
                dst_l = lbuf.at[h % SLOTS, pl.ds(M_HALF, M_HALF)]
            else:
                src_r = rbuf.at[(h - 1) % SLOTS]
                src_l = lbuf.at[(h - 1) % SLOTS]
                dst_r = rbuf.at[h % SLOTS]
                dst_l = lbuf.at[h % SLOTS]
            rdma_r = pltpu.make_async_remote_copy(
                src_ref=src_r, dst_ref=dst_r,
                send_sem=r_send.at[h], recv_sem=r_recv.at[h],
                device_id=(right,), device_id_type=pl.DeviceIdType.MESH)
            rdma_l = pltpu.make_async_remote_copy(
                src_ref=src_l, dst_ref=dst_l,
                send_sem=l_send.at[h], recv_sem=l_recv.at[h],
                device_id=(left,), device_id_type=pl.DeviceIdType.MESH)
            rdma_r.start()
            rdma_l.start()

            if h == 0:
                do_block(x_ref[...], my, 0, M_PER)
            else:
                s = (h - 1) % SLOTS
                do_block(rbuf[s], org_r_ref[h - 1], 0, M_PER)
                do_block(lbuf[s], org_l_ref[h - 1], 0, M_PER)

            rdma_r.wait_recv()
            rdma_r.wait_send()
            rdma_l.wait_recv()
            rdma_l.wait_send()
            if 1 <= h <= H - SLOTS:
                pl.semaphore_signal(r_credit, inc=1, device_id=(left,),
                                    device_id_type=pl.DeviceIdType.MESH)
                pl.semaphore_signal(l_credit, inc=1, device_id=(right,),
                                    device_id_type=pl.DeviceIdType.MESH)

        s = (H - 1) % SLOTS
        do_block(rbuf[s, pl.ds(0, M_HALF)], org_r_ref[H - 1], 0, M_HALF)
        do_block(lbuf[s, pl.ds(M_HALF, M_HALF)], org_l_ref[H - 1],
                 M_HALF, M_HALF)

        amax_buf[...] = jnp.full((8, 128), running, jnp.float32)
        rdmas = []
        for p in range(1, N_DEV):
            peer = jnp.mod(my + p, N_DEV)
            rd = pltpu.make_async_remote_copy(
                src_ref=amax_buf, dst_ref=amax_all.at[p - 1],
                send_sem=a_send.at[p - 1], recv_sem=a_recv.at[p - 1],
                device_id=(peer,), device_id_type=pl.DeviceIdType.MESH)
            rd.start()
            rdmas.append(rd)
        for rd in rdmas:
            rd.wait_send()
        for rd in rdmas:
            rd.wait_recv()
        total = jnp.maximum(running, jnp.max(amax_all[...]))

        scale = total / 448.0
        q = (out_ref[...] / scale).astype(jnp.float8_e4m3fn)
        out_ref[...] = q.astype(jnp.float32) * scale

    smem = pl.BlockSpec(memory_space=pltpu.SMEM)
    return pl.pallas_call(
        body,
        out_shape=jax.ShapeDtypeStruct((N_DEV * m_per, n_per), jnp.float32),
        in_specs=[smem, smem, smem, smem,
                  pl.BlockSpec(memory_space=pltpu.VMEM),
                  pl.BlockSpec(memory_space=pltpu.VMEM)],
        out_specs=pl.BlockSpec(memory_space=pltpu.VMEM),
        scratch_shapes=[
            pltpu.VMEM((SLOTS, M_PER, k), jnp.float32),
            pltpu.VMEM((SLOTS, M_PER, k), jnp.float32),
            pltpu.SemaphoreType.DMA((H,)),
            pltpu.SemaphoreType.DMA((H,)),
            pltpu.SemaphoreType.DMA((H,)),
            pltpu.SemaphoreType.DMA((H,)),
            pltpu.SemaphoreType.REGULAR,
            pltpu.SemaphoreType.REGULAR,
            pltpu.VMEM((8, 128), jnp.float32),
            pltpu.VMEM((N_DEV - 1, 8, 128), jnp.float32),
            pltpu.SemaphoreType.DMA((N_DEV - 1,)),
            pltpu.SemaphoreType.DMA((N_DEV - 1,)),
        ],
        compiler_params=pltpu.CompilerParams(collective_id=0),
    )(right_id, left_id, origins_r, origins_l, x, w_mat)


# device time: 383498 ns/iter; 2.2016x vs baseline; 1.0551x over previous
import numpy as np

import jax
import jax.numpy as jnp
from jax import lax
from jax.experimental import pallas as pl
from jax.experimental.pallas import tpu as pltpu

N_DEV = 32
M_PER = 128
M_HALF = M_PER // 2
SLOTS = 4
H = 16


_COORD_TO_LOGICAL = {}
_l = 0
for _z in range(4):
    for _y in range(4):
        _xs = (1, 0) if _y % 2 else (0, 1)
        for _x in _xs:
            _COORD_TO_LOGICAL[(_x, _y, _z)] = _l
            _l += 1

_PATH_YZ = []
for _z in range(4):
    _ys = range(4) if _z % 2 == 0 else range(3, -1, -1)
    for _y in _ys:
        _PATH_YZ.append((_y, _z))
_RING_COORDS = [(0, y, z) for (y, z) in _PATH_YZ] + \
               [(1, y, z) for (y, z) in reversed(_PATH_YZ)]

RING_ORDER = np.array([_COORD_TO_LOGICAL[c] for c in _RING_COORDS],
                      dtype=np.int32)
RING_POS = np.argsort(RING_ORDER).astype(np.int32)


def kernel(x, w_mat):
    m_per, k = x.shape
    _, n_per = w_mat.shape
    assert m_per == M_PER

    me = lax.axis_index("i")
    ring_order = jnp.asarray(RING_ORDER)
    r_idx = jnp.asarray(RING_POS)[me]
    right_id = ring_order[jnp.mod(r_idx + 1, N_DEV)].reshape(1)
    left_id = ring_order[jnp.mod(r_idx - 1, N_DEV)].reshape(1)
    origins_r = ring_order[jnp.mod(r_idx - 1 - jnp.arange(H), N_DEV)]
    origins_l = ring_order[jnp.mod(r_idx + 1 + jnp.arange(H), N_DEV)]

    def body(right_ref, left_ref, org_r_ref, org_l_ref,
             x_ref, w_ref, out_ref,
             rbuf, lbuf,
             r_send, r_recv, l_send, l_recv,
             r_credit, l_credit,
             amax_buf, amax_all, a_send, a_recv):
        my = lax.axis_index("i")
        right = right_ref[0]
        left = left_ref[0]

        barrier = pltpu.get_barrier_semaphore()
        for nbr in (left, right):
            pl.semaphore_signal(barrier, inc=1, device_id=(nbr,),
                                device_id_type=pl.DeviceIdType.MESH)
        pl.semaphore_wait(barrier, 2)

        def gemm(chunk):
            return lax.dot_general(chunk, w_ref[...],
                                   (((1,), (0,)), ((), ())),
                                   precision=lax.Precision.HIGHEST,
                                   preferred_element_type=jnp.float32)

        running = jnp.float32(0.0)

        def do_block(chunk, origin, row_off, rows):
            nonlocal running
            blk = gemm(chunk)
            out_ref[pl.ds(origin * M_PER + row_off, rows), :] = blk
            running = jnp.maximum(running, jnp.max(jnp.abs(blk)))

        P0 = pl.ds(0, M_HALF)
        P1 = pl.ds(M_HALF, M_HALF)

        def mk(src, dst, sems, h, p, dev):
            return pltpu.make_async_remote_copy(
                src_ref=src, dst_ref=dst,
                send_sem=sems[0].at[h, p], recv_sem=sems[1].at[h, p],
                device_id=(dev,), device_id_type=pl.DeviceIdType.MESH)

        def send_piece(d, h, p):
            buf = rbuf if d == "r" else lbuf
            sems = (r_send, r_recv) if d == "r" else (l_send, l_recv)
            dev = right if d == "r" else left
            pc = P0 if p == 0 else P1
            src = x_ref.at[pc] if h == 0 else buf.at[(h - 1) % SLOTS, pc]
            return mk(src, buf.at[h % SLOTS, pc], sems, h, p, dev)

        live = {}
        for d, p in (("r", 0), ("r", 1), ("l", 0), ("l", 1)):
            rd = send_piece(d, 0, p)
            rd.start()
            live[(d, 0, p)] = rd
        do_block(x_ref[...], my, 0, M_PER)

        for h in range(H):
            last = h == H - 1
            nxt_last = h + 1 == H - 1
            recvs = [("r", 0)] if last else [("r", 0), ("r", 1)]
            if last:
                recvs.append(("l", 1))
            else:
                recvs.extend([("l", 0), ("l", 1)])
            credited = False
            for d, p in recvs:
                live[(d, h, p)].wait_recv()
                if h + 1 < H:
                    if not credited and h + 1 >= SLOTS:
                        pl.semaphore_wait(r_credit, 1)
                        pl.semaphore_wait(l_credit, 1)
                        credited = True
                    if not nxt_last or (d == "r" and p == 0) \
                            or (d == "l" and p == 1):
                        rd = send_piece(d, h + 1, p)
                        rd.start()
                        live[(d, h + 1, p)] = rd
            for key in [k for k in live if k[1] == h]:
                live.pop(key).wait_send()
            if 1 <= h <= H - SLOTS:
                pl.semaphore_signal(r_credit, inc=1, device_id=(left,),
                                    device_id_type=pl.DeviceIdType.MESH)
                pl.semaphore_signal(l_credit, inc=1, device_id=(right,),
                                    device_id_type=pl.DeviceIdType.MESH)
            s = h % SLOTS
            if last:
                do_block(rbuf[s, P0], org_r_ref[h], 0, M_HALF)
                do_block(lbuf[s, P1], org_l_ref[h], M_HALF, M_HALF)
            else:
                do_block(rbuf[s], org_r_ref[h], 0, M_PER)
                do_block(lbuf[s], org_l_ref[h], 0, M_PER)

        amax_buf[...] = jnp.full((8, 128), running, jnp.float32)
        rdmas = []
        for p in range(1, N_DEV):
            peer = jnp.mod(my + p, N_DEV)
            rd = pltpu.make_async_remote_copy(
                src_ref=amax_buf, dst_ref=amax_all.at[p - 1],
                send_sem=a_send.at[p - 1], recv_sem=a_recv.at[p - 1],
                device_id=(peer,), device_id_type=pl.DeviceIdType.MESH)
            rd.start()
            rdmas.append(rd)
        for rd in rdmas:
            rd.wait_send()
        for rd in rdmas:
            rd.wait_recv()
        total = jnp.maximum(running, jnp.max(amax_all[...]))

        scale = total / 448.0
        q = (out_ref[...] / scale).astype(jnp.float8_e4m3fn)
        out_ref[...] = q.astype(jnp.float32) * scale

    smem = pl.BlockSpec(memory_space=pltpu.SMEM)
    return pl.pallas_call(
        body,
        out_shape=jax.ShapeDtypeStruct((N_DEV * m_per, n_per), jnp.float32),
        in_specs=[smem, smem, smem, smem,
                  pl.BlockSpec(memory_space=pltpu.VMEM),
                  pl.BlockSpec(memory_space=pltpu.VMEM)],
        out_specs=pl.BlockSpec(memory_space=pltpu.VMEM),
        scratch_shapes=[
            pltpu.VMEM((SLOTS, M_PER, k), jnp.float32),
            pltpu.VMEM((SLOTS, M_PER, k), jnp.float32),
            pltpu.SemaphoreType.DMA((H, 2)),
            pltpu.SemaphoreType.DMA((H, 2)),
            pltpu.SemaphoreType.DMA((H, 2)),
            pltpu.SemaphoreType.DMA((H, 2)),
            pltpu.SemaphoreType.REGULAR,
            pltpu.SemaphoreType.REGULAR,
            pltpu.VMEM((8, 128), jnp.float32),
            pltpu.VMEM((N_DEV - 1, 8, 128), jnp.float32),
            pltpu.SemaphoreType.DMA((N_DEV - 1,)),
            pltpu.SemaphoreType.DMA((N_DEV - 1,)),
        ],
        compiler_params=pltpu.CompilerParams(collective_id=0),
    )(right_id, left_id, origins_r, origins_l, x, w_mat)


# device time: 377745 ns/iter; 2.2351x vs baseline; 1.0152x over previous
import numpy as np

import jax
import jax.numpy as jnp
from jax import lax
from jax.experimental import pallas as pl
from jax.experimental.pallas import tpu as pltpu

N_DEV = 32
M_PER = 128
M_HALF = M_PER // 2
SLOTS = 4
H = 16


_COORD_TO_LOGICAL = {}
_l = 0
for _z in range(4):
    for _y in range(4):
        _xs = (1, 0) if _y % 2 else (0, 1)
        for _x in _xs:
            _COORD_TO_LOGICAL[(_x, _y, _z)] = _l
            _l += 1

_PATH_YZ = []
for _z in range(4):
    _ys = range(4) if _z % 2 == 0 else range(3, -1, -1)
    for _y in _ys:
        _PATH_YZ.append((_y, _z))
_RING_COORDS = [(0, y, z) for (y, z) in _PATH_YZ] + \
               [(1, y, z) for (y, z) in reversed(_PATH_YZ)]

RING_ORDER = np.array([_COORD_TO_LOGICAL[c] for c in _RING_COORDS],
                      dtype=np.int32)
RING_POS = np.argsort(RING_ORDER).astype(np.int32)


def kernel(x, w_mat):
    m_per, k = x.shape
    _, n_per = w_mat.shape
    assert m_per == M_PER

    def ring_pos_of(l):
        z = l // 8
        rem = l % 8
        y = rem // 2
        xp = rem % 2
        x = jnp.where(y % 2 == 0, xp, 1 - xp)
        i = 4 * z + jnp.where(z % 2 == 0, y, 3 - y)
        return jnp.where(x == 0, i, N_DEV - 1 - i)

    def ring_order_of(r):
        x = (r >= N_DEV // 2).astype(jnp.int32)
        i = jnp.where(x == 0, r, N_DEV - 1 - r)
        z = i // 4
        yy = i % 4
        y = jnp.where(z % 2 == 0, yy, 3 - yy)
        xr = jnp.where(y % 2 == 0, x, 1 - x)
        return 8 * z + 2 * y + xr

    me = lax.axis_index("i")
    r_idx = ring_pos_of(me)
    right_id = ring_order_of(jnp.mod(r_idx + 1, N_DEV)).reshape(1)
    left_id = ring_order_of(jnp.mod(r_idx - 1, N_DEV)).reshape(1)
    origins_r = ring_order_of(jnp.mod(r_idx - 1 - jnp.arange(H), N_DEV))
    origins_l = ring_order_of(jnp.mod(r_idx + 1 + jnp.arange(H), N_DEV))

    def body(right_ref, left_ref, org_r_ref, org_l_ref,
             x_ref, w_ref, out_ref,
             rbuf, lbuf,
             r_send, r_recv, l_send, l_recv,
             r_credit, l_credit,
             amax_buf, amax_all, a_send, a_recv):
        my = lax.axis_index("i")
        right = right_ref[0]
        left = left_ref[0]

        barrier = pltpu.get_barrier_semaphore()
        for nbr in (left, right):
            pl.semaphore_signal(barrier, inc=1, device_id=(nbr,),
                                device_id_type=pl.DeviceIdType.MESH)
        pl.semaphore_wait(barrier, 2)

        def gemm(chunk):
            return lax.dot_general(chunk, w_ref[...],
                                   (((1,), (0,)), ((), ())),
                                   precision=lax.Precision.HIGHEST,
                                   preferred_element_type=jnp.float32)

        running = jnp.float32(0.0)

        def do_block(chunk, origin, row_off, rows):
            nonlocal running
            blk = gemm(chunk)
            out_ref[pl.ds(origin * M_PER + row_off, rows), :] = blk
            running = jnp.maximum(running, jnp.max(jnp.abs(blk)))

        P0 = pl.ds(0, M_HALF)
        P1 = pl.ds(M_HALF, M_HALF)

        def mk(src, dst, sems, h, p, dev):
            return pltpu.make_async_remote_copy(
                src_ref=src, dst_ref=dst,
                send_sem=sems[0].at[h, p], recv_sem=sems[1].at[h, p],
                device_id=(dev,), device_id_type=pl.DeviceIdType.MESH)

        def send_piece(d, h, p):
            buf = rbuf if d == "r" else lbuf
            sems = (r_send, r_recv) if d == "r" else (l_send, l_recv)
            dev = right if d == "r" else left
            pc = P0 if p == 0 else P1
            src = x_ref.at[pc] if h == 0 else buf.at[(h - 1) % SLOTS, pc]
            return mk(src, buf.at[h % SLOTS, pc], sems, h, p, dev)

        live = {}
        for d, p in (("r", 0), ("r", 1), ("l", 0), ("l", 1)):
            rd = send_piece(d, 0, p)
            rd.start()
            live[(d, 0, p)] = rd
        do_block(x_ref[...], my, 0, M_PER)

        for h in range(H):
            last = h == H - 1
            nxt_last = h + 1 == H - 1
            recvs = [("r", 0)] if last else [("r", 0), ("r", 1)]
            if last:
                recvs.append(("l", 1))
            else:
                recvs.extend([("l", 0), ("l", 1)])
            credited = False
            for d, p in recvs:
                live[(d, h, p)].wait_recv()
                if h + 1 < H:
                    if not credited and h + 1 >= SLOTS:
                        pl.semaphore_wait(r_credit, 1)
                        pl.semaphore_wait(l_credit, 1)
                        credited = True
                    if not nxt_last or (d == "r" and p == 0) \
                            or (d == "l" and p == 1):
                        rd = send_piece(d, h + 1, p)
                        rd.start()
                        live[(d, h + 1, p)] = rd
            for key in [k for k in live if k[1] == h]:
                live.pop(key).wait_send()
            if 1 <= h <= H - SLOTS:
                pl.semaphore_signal(r_credit, inc=1, device_id=(left,),
                                    device_id_type=pl.DeviceIdType.MESH)
                pl.semaphore_signal(l_credit, inc=1, device_id=(right,),
                                    device_id_type=pl.DeviceIdType.MESH)
            s = h % SLOTS
            if last:
                do_block(rbuf[s, P0], org_r_ref[h], 0, M_HALF)
                do_block(lbuf[s, P1], org_l_ref[h], M_HALF, M_HALF)
            else:
                do_block(rbuf[s], org_r_ref[h], 0, M_PER)
                do_block(lbuf[s], org_l_ref[h], 0, M_PER)

        amax_buf[...] = jnp.full((8, 128), running, jnp.float32)
        rdmas = []
        for p in range(1, N_DEV):
            peer = jnp.mod(my + p, N_DEV)
            rd = pltpu.make_async_remote_copy(
                src_ref=amax_buf, dst_ref=amax_all.at[p - 1],
                send_sem=a_send.at[p - 1], recv_sem=a_recv.at[p - 1],
                device_id=(peer,), device_id_type=pl.DeviceIdType.MESH)
            rd.start()
            rdmas.append(rd)
        for rd in rdmas:
            rd.wait_send()
        for rd in rdmas:
            rd.wait_recv()
        total = jnp.maximum(running, jnp.max(amax_all[...]))

        scale = total / 448.0
        q = (out_ref[...] / scale).astype(jnp.float8_e4m3fn)
        out_ref[...] = q.astype(jnp.float32) * scale

    smem = pl.BlockSpec(memory_space=pltpu.SMEM)
    return pl.pallas_call(
        body,
        out_shape=jax.ShapeDtypeStruct((N_DEV * m_per, n_per), jnp.float32),
        in_specs=[smem, smem, smem, smem,
                  pl.BlockSpec(memory_space=pltpu.VMEM),
                  pl.BlockSpec(memory_space=pltpu.VMEM)],
        out_specs=pl.BlockSpec(memory_space=pltpu.VMEM),
        scratch_shapes=[
            pltpu.VMEM((SLOTS, M_PER, k), jnp.float32),
            pltpu.VMEM((SLOTS, M_PER, k), jnp.float32),
            pltpu.SemaphoreType.DMA((H, 2)),
            pltpu.SemaphoreType.DMA((H, 2)),
            pltpu.SemaphoreType.DMA((H, 2)),
            pltpu.SemaphoreType.DMA((H, 2)),
            pltpu.SemaphoreType.REGULAR,
            pltpu.SemaphoreType.REGULAR,
            pltpu.VMEM((8, 128), jnp.float32),
            pltpu.VMEM((N_DEV - 1, 8, 128), jnp.float32),
            pltpu.SemaphoreType.DMA((N_DEV - 1,)),
            pltpu.SemaphoreType.DMA((N_DEV - 1,)),
        ],
        compiler_params=pltpu.CompilerParams(collective_id=0),
    )(right_id, left_id, origins_r, origins_l, x, w_mat)
